# baseline (speedup 1.0000x reference)
import functools

import jax
import jax.numpy as jnp
from jax import lax
from jax.experimental import pallas as pl
from jax.experimental.pallas import tpu as pltpu
from jax.experimental.pallas import tpu_sc as plsc

D_MODEL = 256
DIM_FF = 2048
NHEAD = 8
DH = D_MODEL // NHEAD
NPOINTS = 4
B = 4
LQ = 1024
H = 128
W = 128
LIN = H * W

NC = 2
NS = 16
NW = NC * NS

CQ = 64
NCHUNK = LQ // CQ
NROW = CQ * NPOINTS * 4
NIDX = NROW // 128


def _nt_dot(x, w):
    return lax.dot_general(x, w, (((1,), (1,)), ((), ())),
                           preferred_element_type=jnp.float32)


def _proj_body(xa_ref, wa_ref, ba_ref, wb_ref, bb_ref,
               t_ref, p_ref, wq_ref, bq_ref, o_ref, oq_ref):
    x = xa_ref[...]
    a = _nt_dot(x, wa_ref[...]) + ba_ref[...]
    bpl = _nt_dot(x, wb_ref[...]) + bb_ref[...]
    packed = pltpu.pack_elementwise([a, bpl], packed_dtype=jnp.bfloat16)
    o_ref[...] = lax.bitcast_convert_type(packed, jnp.float32)
    q = t_ref[...] + p_ref[...]
    oq_ref[...] = _nt_dot(q, wq_ref[...]) + bq_ref[...]


def _layernorm(x, g, b):
    mu = jnp.mean(x, axis=-1, keepdims=True)
    xc = x - mu
    var = jnp.mean(xc * xc, axis=-1, keepdims=True)
    return xc * lax.rsqrt(var + 1e-5) * g + b


def _ffn_body(s_ref, s2_ref, t_ref, wo_ref, bo_ref, w1_ref, b1_ref, w2_ref,
              b2_ref, g1_ref, be1_ref, g2_ref, be2_ref, o_ref):
    s = jnp.concatenate([s_ref[...], s2_ref[...]], axis=-1)
    tgt2 = _nt_dot(s, wo_ref[...]) + bo_ref[...]
    x = _layernorm(t_ref[...] + tgt2, g1_ref[...], be1_ref[...])
    ff = jnp.maximum(_nt_dot(x, w1_ref[...]) + b1_ref[...], 0.0)
    ff2 = _nt_dot(ff, w2_ref[...]) + b2_ref[...]
    o_ref[...] = _layernorm(x + ff2, g2_ref[...], be2_ref[...])


def _full(shape):
    return pl.BlockSpec(shape, lambda i: (0,) * len(shape))


def _rows(blk, cols):
    return pl.BlockSpec((blk, cols), lambda i: (i, 0))


def _sc_body(value_hbm, par_hbm, ref_hbm, out_lo_hbm, out_hi_hbm,
             par_v, ref_v, idx_v, w_v, rows_v, out_v, sem, psem):
    cid = lax.axis_index("c")
    sid = lax.axis_index("s")
    wid = sid * NC + cid
    b = wid // NHEAD
    h = wid % NHEAD
    qi16 = lax.broadcasted_iota(jnp.int32, (16,), 0)
    tbase = b * (LIN * NHEAD) + h

    def fire_par(ci):
        p1 = ci & 1
        q0 = pl.multiple_of(ci * CQ, CQ)
        pltpu.async_copy(par_hbm.at[b, pl.ds(q0 * 128, CQ * 128)],
                         par_v.at[p1], psem)
        pltpu.async_copy(ref_hbm.at[b, pl.ds(q0 * 2, CQ * 2)],
                         ref_v.at[p1, pl.ds(0, CQ * 2)], psem)

    def drain_par(ci):
        p1 = ci & 1
        pltpu.make_async_copy(par_hbm.at[b, pl.ds(0, CQ * 128)],
                              par_v.at[p1], psem).wait()
        pltpu.make_async_copy(ref_hbm.at[b, pl.ds(0, CQ * 2)],
                              ref_v.at[p1, pl.ds(0, CQ * 2)], psem).wait()

    def gen_and_fire(ci):
        p1 = ci & 1
        p1v = jnp.full((16,), p1, jnp.int32)
        for g in range(CQ // 16):
            qi = qi16 + (g * 16)
            base = qi * 128
            rx = plsc.load_gather(ref_v, [p1v, qi * 2])
            ry = plsc.load_gather(ref_v, [p1v, qi * 2 + 1])
            gxb = rx * float(W) - 0.5
            gyb = ry * float(H) - 0.5
            logits = [plsc.load_gather(par_v, [p1v, base + (64 + h * 4 + p)])
                      for p in range(NPOINTS)]
            m = jnp.maximum(jnp.maximum(logits[0], logits[1]),
                            jnp.maximum(logits[2], logits[3]))
            es = [jnp.exp(l - m) for l in logits]
            inv = 1.0 / (es[0] + es[1] + es[2] + es[3])
            for p in range(NPOINTS):
                ap = es[p] * inv
                ox = plsc.load_gather(par_v, [p1v, base + h * 8 + 2 * p])
                oy = plsc.load_gather(par_v,
                                      [p1v, base + h * 8 + 2 * p + 1])
                gx = gxb + ox
                gy = gyb + oy
                xt = gx.astype(jnp.int32)
                xtf = xt.astype(jnp.float32)
                xneg = xtf > gx
                x0 = jnp.where(xneg, xt - 1, xt)
                fx = gx - jnp.where(xneg, xtf - 1.0, xtf)
                yt = gy.astype(jnp.int32)
                ytf = yt.astype(jnp.float32)
                yneg = ytf > gy
                y0 = jnp.where(yneg, yt - 1, yt)
                fy = gy - jnp.where(yneg, ytf - 1.0, ytf)
                wx = [1.0 - fx, fx]
                wy = [1.0 - fy, fy]
                for dy in range(2):
                    yc = y0 + dy
                    vy = (yc >= 0) & (yc <= H - 1)
                    yci = jnp.clip(yc, 0, H - 1)
                    for dx in range(2):
                        xc = x0 + dx
                        ok = (xc >= 0) & (xc <= W - 1) & vy
                        xci = jnp.clip(xc, 0, W - 1)
                        gidx = tbase + (yci * W + xci) * NHEAD
                        wt = ap * wx[dx] * wy[dy] * jnp.where(ok, 1.0, 0.0)
                        pos = qi * 16 + (p * 4 + dy * 2 + dx)
                        plsc.store_scatter(
                            idx_v, [p1v, lax.shift_right_logical(pos, 7),
                                    lax.bitwise_and(pos, 127)], gidx)
                        plsc.store_scatter(w_v, [p1v, pos], wt)
        for i in range(NIDX):
            pltpu.async_copy(value_hbm.at[idx_v.at[p1, i]],
                             rows_v.at[p1, pl.ds(i * 128, 128)], sem)

    fire_par(0)
    fire_par(1)

    def chunk_body(ci, carry):
        @pl.when(ci < NCHUNK)
        def _():
            drain_par(ci)
            gen_and_fire(ci)

            @pl.when(ci + 2 < NCHUNK)
            def _():
                fire_par(ci + 2)

        @pl.when(ci >= 1)
        def _():
            cj = ci - 1
            p0 = cj & 1
            q0 = pl.multiple_of(cj * CQ, CQ)
            pltpu.make_async_copy(value_hbm.at[pl.ds(0, NROW)],
                                  rows_v.at[p0], sem).wait()

            def q_body(q, carry2):
                rbase = q * 16
                wvec = w_v[p0, pl.ds(rbase, 16)]
                acc0 = jnp.zeros((16,), jnp.float32)
                acc1 = jnp.zeros((16,), jnp.float32)
                for j in range(16):
                    wj = wvec[j]
                    row = plsc.bitcast(rows_v[p0, rbase + j, :],
                                       jnp.bfloat16)
                    ev, od = plsc.unpack(row,
                                         format=plsc.PackFormat.INTERLEAVED)
                    acc0 = acc0 + ev * wj
                    acc1 = acc1 + od * wj
                out_v[q, pl.ds(0, 16)] = acc0
                out_v[q, pl.ds(16, 16)] = acc1
                return carry2

            lax.fori_loop(0, CQ, q_body, 0, unroll=False)
            lane0 = pl.multiple_of((h % 4) * DH, DH)
            row0 = pl.multiple_of(b * LQ + q0, CQ)

            @pl.when(h < 4)
            def _():
                pltpu.sync_copy(
                    out_v, out_lo_hbm.at[pl.ds(row0, CQ), pl.ds(lane0, DH)])

            @pl.when(h >= 4)
            def _():
                pltpu.sync_copy(
                    out_v, out_hi_hbm.at[pl.ds(row0, CQ), pl.ds(lane0, DH)])

        return carry

    lax.fori_loop(0, NCHUNK + 1, chunk_body, 0, unroll=False)


@jax.jit
def _run(tgt, uv_feature, reference_points, query_pos,
         W_off, b_off, W_attn, b_attn, W_val, b_val, W_out, b_out,
         W1, b1, W2, b2, g1, be1, g2, be2):
    f32 = jnp.float32
    uv2d = uv_feature.reshape(B * LIN, D_MODEL)
    tgt2d = tgt.reshape(B * LQ, D_MODEL)
    qp2d = query_pos.reshape(B * LQ, D_MODEL)

    W_oa = jnp.concatenate(
        [W_off, W_attn, jnp.zeros((32, D_MODEL), f32)], axis=0)
    b_oa = jnp.concatenate([b_off, b_attn, jnp.zeros((32,), f32)])
    Wv = W_val.reshape(NHEAD, DH, D_MODEL)
    Wv_a = Wv[:, :16].reshape(NHEAD * 16, D_MODEL)
    Wv_b = Wv[:, 16:].reshape(NHEAD * 16, D_MODEL)
    bv = b_val.reshape(NHEAD, DH)
    bv_a = bv[:, :16].reshape(1, NHEAD * 16)
    bv_b = bv[:, 16:].reshape(1, NHEAD * 16)
    VBLK = 2048
    QBLK = 128
    NSTEP = B * LIN // VBLK
    value2d, params = pl.pallas_call(
        _proj_body,
        grid=(NSTEP,),
        in_specs=[_rows(VBLK, D_MODEL),
                  _full((128, D_MODEL)), _full((1, 128)),
                  _full((128, D_MODEL)), _full((1, 128)),
                  _rows(QBLK, D_MODEL), _rows(QBLK, D_MODEL),
                  _full((128, D_MODEL)), _full((1, 128))],
        out_specs=(_rows(VBLK, 128), _rows(QBLK, 128)),
        out_shape=(jax.ShapeDtypeStruct((B * LIN, 128), f32),
                   jax.ShapeDtypeStruct((B * LQ, 128), f32)),
    )(uv2d, Wv_a, bv_a, Wv_b, bv_b,
      tgt2d, qp2d, W_oa, b_oa.reshape(1, 128))

    table = value2d.reshape(B * LIN * NHEAD, 16)
    par2d = params.reshape(B, LQ * 128)
    ref2d = reference_points.reshape(B, LQ * 2)
    mesh = plsc.VectorSubcoreMesh(core_axis_name="c", subcore_axis_name="s",
                                  num_cores=NC, num_subcores=NS)
    sampled_lo, sampled_hi = pl.kernel(
        _sc_body,
        out_type=(jax.ShapeDtypeStruct((B * LQ, 128), f32),
                  jax.ShapeDtypeStruct((B * LQ, 128), f32)),
        mesh=mesh,
        compiler_params=pltpu.CompilerParams(needs_layout_passes=False,
                                             use_tc_tiling_on_sc=False),
        scratch_types=[
            pltpu.VMEM((2, CQ * 128), f32),
            pltpu.VMEM((2, CQ * 2 + 128), f32),
            pltpu.VMEM((2, NIDX, 128), jnp.int32),
            pltpu.VMEM((2, NROW), f32),
            pltpu.VMEM((2, NROW, 16), f32),
            pltpu.VMEM((CQ, DH), f32),
            pltpu.SemaphoreType.DMA,
            pltpu.SemaphoreType.DMA,
        ],
    )(table, par2d, ref2d)

    FBLK = 512
    out = pl.pallas_call(
        _ffn_body,
        grid=(B * LQ // FBLK,),
        in_specs=[_rows(FBLK, 128), _rows(FBLK, 128),
                  _rows(FBLK, D_MODEL),
                  _full((D_MODEL, D_MODEL)), _full((1, D_MODEL)),
                  _full((DIM_FF, D_MODEL)), _full((1, DIM_FF)),
                  _full((D_MODEL, DIM_FF)), _full((1, D_MODEL)),
                  _full((1, D_MODEL)), _full((1, D_MODEL)),
                  _full((1, D_MODEL)), _full((1, D_MODEL))],
        out_specs=_rows(FBLK, D_MODEL),
        out_shape=jax.ShapeDtypeStruct((B * LQ, D_MODEL), f32),
    )(sampled_lo, sampled_hi, tgt2d,
      W_out, b_out.reshape(1, D_MODEL),
      W1, b1.reshape(1, DIM_FF),
      W2, b2.reshape(1, D_MODEL),
      g1.reshape(1, D_MODEL), be1.reshape(1, D_MODEL),
      g2.reshape(1, D_MODEL), be2.reshape(1, D_MODEL))
    return out.reshape(B, LQ, D_MODEL)


def kernel(tgt, uv_feature, reference_points, query_pos,
           input_spatial_shapes, input_level_start_index,
           W_off, b_off, W_attn, b_attn, W_val, b_val, W_out, b_out,
           W1, b1, W2, b2, g1, be1, g2, be2):
    return _run(tgt, uv_feature, reference_points, query_pos,
                W_off, b_off, W_attn, b_attn, W_val, b_val, W_out, b_out,
                W1, b1, W2, b2, g1, be1, g2, be2)


# speedup vs baseline: 1.0230x; 1.0230x over previous
import functools

import jax
import jax.numpy as jnp
from jax import lax
from jax.experimental import pallas as pl
from jax.experimental.pallas import tpu as pltpu
from jax.experimental.pallas import tpu_sc as plsc

D_MODEL = 256
DIM_FF = 2048
NHEAD = 8
DH = D_MODEL // NHEAD
NPOINTS = 4
B = 4
LQ = 1024
H = 128
W = 128
LIN = H * W

NC = 2
NS = 16
NW = NC * NS

CQ = 64
NCHUNK = LQ // CQ
NROW = CQ * NPOINTS * 4
NIDX = NROW // 128


def _nt_dot(x, w):
    return lax.dot_general(x, w, (((1,), (1,)), ((), ())),
                           preferred_element_type=jnp.float32)


def _proj_body(x_ref, w_ref, b_ref, t_ref, p_ref, wq_ref, bq_ref,
               o_ref, oq_ref):
    r = _nt_dot(x_ref[...], w_ref[...]) + b_ref[...]
    o_ref[0] = r[:, :128]
    o_ref[1] = r[:, 128:]
    q = t_ref[...] + p_ref[...]
    oq_ref[...] = _nt_dot(q, wq_ref[...]) + bq_ref[...]


def _layernorm(x, g, b):
    mu = jnp.mean(x, axis=-1, keepdims=True)
    xc = x - mu
    var = jnp.mean(xc * xc, axis=-1, keepdims=True)
    return xc * lax.rsqrt(var + 1e-5) * g + b


def _ffn_body(s_ref, s2_ref, t_ref, wo_ref, bo_ref, w1_ref, b1_ref, w2_ref,
              b2_ref, g1_ref, be1_ref, g2_ref, be2_ref, o_ref):
    s = jnp.concatenate([s_ref[...], s2_ref[...]], axis=-1)
    tgt2 = _nt_dot(s, wo_ref[...]) + bo_ref[...]
    x = _layernorm(t_ref[...] + tgt2, g1_ref[...], be1_ref[...])
    ff = jnp.maximum(_nt_dot(x, w1_ref[...]) + b1_ref[...], 0.0)
    ff2 = _nt_dot(ff, w2_ref[...]) + b2_ref[...]
    o_ref[...] = _layernorm(x + ff2, g2_ref[...], be2_ref[...])


def _full(shape):
    return pl.BlockSpec(shape, lambda i: (0,) * len(shape))


def _rows(blk, cols):
    return pl.BlockSpec((blk, cols), lambda i: (i, 0))


def _sc_body(value_hbm, par_hbm, ref_hbm, out_lo_hbm, out_hi_hbm,
             par_v, ref_v, idx_v, w_v, rows_v, out_v, sem, psem):
    cid = lax.axis_index("c")
    sid = lax.axis_index("s")
    wid = sid * NC + cid
    b = wid // NHEAD
    h = wid % NHEAD
    qi16 = lax.broadcasted_iota(jnp.int32, (16,), 0)
    tbase = (h // 4) * (B * LIN * 4) + b * (LIN * 4) + (h % 4)

    def fire_par(ci):
        p1 = ci & 1
        q0 = pl.multiple_of(ci * CQ, CQ)
        pltpu.async_copy(par_hbm.at[b, pl.ds(q0 * 128, CQ * 128)],
                         par_v.at[p1], psem)
        pltpu.async_copy(ref_hbm.at[b, pl.ds(q0 * 2, CQ * 2)],
                         ref_v.at[p1, pl.ds(0, CQ * 2)], psem)

    def drain_par(ci):
        p1 = ci & 1
        pltpu.make_async_copy(par_hbm.at[b, pl.ds(0, CQ * 128)],
                              par_v.at[p1], psem).wait()
        pltpu.make_async_copy(ref_hbm.at[b, pl.ds(0, CQ * 2)],
                              ref_v.at[p1, pl.ds(0, CQ * 2)], psem).wait()

    def gen_and_fire(ci):
        p1 = ci & 1
        p1v = jnp.full((16,), p1, jnp.int32)
        for g in range(CQ // 16):
            qi = qi16 + (g * 16)
            base = qi * 128
            rx = plsc.load_gather(ref_v, [p1v, qi * 2])
            ry = plsc.load_gather(ref_v, [p1v, qi * 2 + 1])
            gxb = rx * float(W) - 0.5
            gyb = ry * float(H) - 0.5
            logits = [plsc.load_gather(par_v, [p1v, base + (64 + h * 4 + p)])
                      for p in range(NPOINTS)]
            m = jnp.maximum(jnp.maximum(logits[0], logits[1]),
                            jnp.maximum(logits[2], logits[3]))
            es = [jnp.exp(l - m) for l in logits]
            inv = 1.0 / (es[0] + es[1] + es[2] + es[3])
            for p in range(NPOINTS):
                ap = es[p] * inv
                ox = plsc.load_gather(par_v, [p1v, base + h * 8 + 2 * p])
                oy = plsc.load_gather(par_v,
                                      [p1v, base + h * 8 + 2 * p + 1])
                gx = gxb + ox
                gy = gyb + oy
                xt = gx.astype(jnp.int32)
                xtf = xt.astype(jnp.float32)
                xneg = xtf > gx
                x0 = jnp.where(xneg, xt - 1, xt)
                fx = gx - jnp.where(xneg, xtf - 1.0, xtf)
                yt = gy.astype(jnp.int32)
                ytf = yt.astype(jnp.float32)
                yneg = ytf > gy
                y0 = jnp.where(yneg, yt - 1, yt)
                fy = gy - jnp.where(yneg, ytf - 1.0, ytf)
                wx = [1.0 - fx, fx]
                wy = [1.0 - fy, fy]
                for dy in range(2):
                    yc = y0 + dy
                    vy = (yc >= 0) & (yc <= H - 1)
                    yci = jnp.clip(yc, 0, H - 1)
                    for dx in range(2):
                        xc = x0 + dx
                        ok = (xc >= 0) & (xc <= W - 1) & vy
                        xci = jnp.clip(xc, 0, W - 1)
                        gidx = tbase + (yci * W + xci) * 4
                        wt = ap * wx[dx] * wy[dy] * jnp.where(ok, 1.0, 0.0)
                        pos = qi * 16 + (p * 4 + dy * 2 + dx)
                        plsc.store_scatter(
                            idx_v, [p1v, lax.shift_right_logical(pos, 7),
                                    lax.bitwise_and(pos, 127)], gidx)
                        plsc.store_scatter(w_v, [p1v, pos], wt)
        for i in range(NIDX):
            pltpu.async_copy(value_hbm.at[idx_v.at[p1, i]],
                             rows_v.at[p1, pl.ds(i * 128, 128)], sem)

    fire_par(0)
    fire_par(1)

    def chunk_body(ci, carry):
        @pl.when(ci < NCHUNK)
        def _():
            drain_par(ci)
            gen_and_fire(ci)

            @pl.when(ci + 2 < NCHUNK)
            def _():
                fire_par(ci + 2)

        @pl.when(ci >= 1)
        def _():
            cj = ci - 1
            p0 = cj & 1
            q0 = pl.multiple_of(cj * CQ, CQ)
            pltpu.make_async_copy(value_hbm.at[pl.ds(0, NROW)],
                                  rows_v.at[p0], sem).wait()

            def q_body(q, carry2):
                rbase = q * 16
                wvec = w_v[p0, pl.ds(rbase, 16)]
                acc0 = jnp.zeros((16,), jnp.float32)
                acc1 = jnp.zeros((16,), jnp.float32)
                for j in range(16):
                    wj = wvec[j]
                    acc0 = acc0 + rows_v[p0, rbase + j, pl.ds(0, 16)] * wj
                    acc1 = acc1 + rows_v[p0, rbase + j, pl.ds(16, 16)] * wj
                out_v[q, pl.ds(0, 16)] = acc0
                out_v[q, pl.ds(16, 16)] = acc1
                return carry2

            lax.fori_loop(0, CQ, q_body, 0, unroll=False)
            lane0 = pl.multiple_of((h % 4) * DH, DH)
            row0 = pl.multiple_of(b * LQ + q0, CQ)

            @pl.when(h < 4)
            def _():
                pltpu.sync_copy(
                    out_v, out_lo_hbm.at[pl.ds(row0, CQ), pl.ds(lane0, DH)])

            @pl.when(h >= 4)
            def _():
                pltpu.sync_copy(
                    out_v, out_hi_hbm.at[pl.ds(row0, CQ), pl.ds(lane0, DH)])

        return carry

    lax.fori_loop(0, NCHUNK + 1, chunk_body, 0, unroll=False)


@jax.jit
def _run(tgt, uv_feature, reference_points, query_pos,
         W_off, b_off, W_attn, b_attn, W_val, b_val, W_out, b_out,
         W1, b1, W2, b2, g1, be1, g2, be2):
    f32 = jnp.float32
    uv2d = uv_feature.reshape(B * LIN, D_MODEL)
    tgt2d = tgt.reshape(B * LQ, D_MODEL)
    qp2d = query_pos.reshape(B * LQ, D_MODEL)

    W_oa = jnp.concatenate(
        [W_off, W_attn, jnp.zeros((32, D_MODEL), f32)], axis=0)
    b_oa = jnp.concatenate([b_off, b_attn, jnp.zeros((32,), f32)])
    VBLK = 2048
    QBLK = 128
    NSTEP = B * LIN // VBLK
    value2d, params = pl.pallas_call(
        _proj_body,
        grid=(NSTEP,),
        in_specs=[_rows(VBLK, D_MODEL), _full((D_MODEL, D_MODEL)),
                  _full((1, D_MODEL)),
                  _rows(QBLK, D_MODEL), _rows(QBLK, D_MODEL),
                  _full((128, D_MODEL)), _full((1, 128))],
        out_specs=(pl.BlockSpec((2, VBLK, 128), lambda i: (0, i, 0)),
                   _rows(QBLK, 128)),
        out_shape=(jax.ShapeDtypeStruct((2, B * LIN, 128), f32),
                   jax.ShapeDtypeStruct((B * LQ, 128), f32)),
    )(uv2d, W_val, b_val.reshape(1, D_MODEL),
      tgt2d, qp2d, W_oa, b_oa.reshape(1, 128))

    table = value2d.reshape(B * LIN * NHEAD, DH)
    par2d = params.reshape(B, LQ * 128)
    ref2d = reference_points.reshape(B, LQ * 2)
    mesh = plsc.VectorSubcoreMesh(core_axis_name="c", subcore_axis_name="s",
                                  num_cores=NC, num_subcores=NS)
    sampled_lo, sampled_hi = pl.kernel(
        _sc_body,
        out_type=(jax.ShapeDtypeStruct((B * LQ, 128), f32),
                  jax.ShapeDtypeStruct((B * LQ, 128), f32)),
        mesh=mesh,
        compiler_params=pltpu.CompilerParams(needs_layout_passes=False,
                                             use_tc_tiling_on_sc=False),
        scratch_types=[
            pltpu.VMEM((2, CQ * 128), f32),
            pltpu.VMEM((2, CQ * 2 + 128), f32),
            pltpu.VMEM((2, NIDX, 128), jnp.int32),
            pltpu.VMEM((2, NROW), f32),
            pltpu.VMEM((2, NROW, DH), f32),
            pltpu.VMEM((CQ, DH), f32),
            pltpu.SemaphoreType.DMA,
            pltpu.SemaphoreType.DMA,
        ],
    )(table, par2d, ref2d)

    FBLK = 512
    out = pl.pallas_call(
        _ffn_body,
        grid=(B * LQ // FBLK,),
        in_specs=[_rows(FBLK, 128), _rows(FBLK, 128),
                  _rows(FBLK, D_MODEL),
                  _full((D_MODEL, D_MODEL)), _full((1, D_MODEL)),
                  _full((DIM_FF, D_MODEL)), _full((1, DIM_FF)),
                  _full((D_MODEL, DIM_FF)), _full((1, D_MODEL)),
                  _full((1, D_MODEL)), _full((1, D_MODEL)),
                  _full((1, D_MODEL)), _full((1, D_MODEL))],
        out_specs=_rows(FBLK, D_MODEL),
        out_shape=jax.ShapeDtypeStruct((B * LQ, D_MODEL), f32),
    )(sampled_lo, sampled_hi, tgt2d,
      W_out, b_out.reshape(1, D_MODEL),
      W1, b1.reshape(1, DIM_FF),
      W2, b2.reshape(1, D_MODEL),
      g1.reshape(1, D_MODEL), be1.reshape(1, D_MODEL),
      g2.reshape(1, D_MODEL), be2.reshape(1, D_MODEL))
    return out.reshape(B, LQ, D_MODEL)


def kernel(tgt, uv_feature, reference_points, query_pos,
           input_spatial_shapes, input_level_start_index,
           W_off, b_off, W_attn, b_attn, W_val, b_val, W_out, b_out,
           W1, b1, W2, b2, g1, be1, g2, be2):
    return _run(tgt, uv_feature, reference_points, query_pos,
                W_off, b_off, W_attn, b_attn, W_val, b_val, W_out, b_out,
                W1, b1, W2, b2, g1, be1, g2, be2)


# speedup vs baseline: 1.0664x; 1.0425x over previous
import functools

import jax
import jax.numpy as jnp
from jax import lax
from jax.experimental import pallas as pl
from jax.experimental.pallas import tpu as pltpu
from jax.experimental.pallas import tpu_sc as plsc

D_MODEL = 256
DIM_FF = 2048
NHEAD = 8
DH = D_MODEL // NHEAD
NPOINTS = 4
B = 4
LQ = 1024
H = 128
W = 128
LIN = H * W

NC = 2
NS = 16
NW = NC * NS

CQ = 64
NCHUNK = LQ // CQ
NROW = CQ * NPOINTS * 4
NIDX = NROW // 128


def _nt_dot(x, w):
    return lax.dot_general(x, w, (((1,), (1,)), ((), ())),
                           preferred_element_type=jnp.float32)


def _proj_body(x_ref, w_ref, b_ref, t_ref, p_ref, wq_ref, bq_ref,
               o_ref, oq_ref):
    r = _nt_dot(x_ref[...], w_ref[...]) + b_ref[...]
    o_ref[0] = r[:, :128]
    o_ref[1] = r[:, 128:]
    q = t_ref[...] + p_ref[...]
    oq_ref[...] = _nt_dot(q, wq_ref[...]) + bq_ref[...]


def _layernorm(x, g, b):
    mu = jnp.mean(x, axis=-1, keepdims=True)
    xc = x - mu
    var = jnp.mean(xc * xc, axis=-1, keepdims=True)
    return xc * lax.rsqrt(var + 1e-5) * g + b


def _ffn_body(s_ref, s2_ref, t_ref, wo_ref, bo_ref, w1_ref, b1_ref, w2_ref,
              b2_ref, g1_ref, be1_ref, g2_ref, be2_ref, o_ref):
    s = jnp.concatenate([s_ref[...], s2_ref[...]], axis=-1)
    tgt2 = _nt_dot(s, wo_ref[...]) + bo_ref[...]
    x = _layernorm(t_ref[...] + tgt2, g1_ref[...], be1_ref[...])
    ff = jnp.maximum(_nt_dot(x, w1_ref[...]) + b1_ref[...], 0.0)
    ff2 = _nt_dot(ff, w2_ref[...]) + b2_ref[...]
    o_ref[...] = _layernorm(x + ff2, g2_ref[...], be2_ref[...])


def _full(shape):
    return pl.BlockSpec(shape, lambda i: (0,) * len(shape))


def _rows(blk, cols):
    return pl.BlockSpec((blk, cols), lambda i: (i, 0))


def _sc_body(value_hbm, par_hbm, ref_hbm, out_lo_hbm, out_hi_hbm,
             par_v, ref_v, idx_v, w_v, rows_v, out_v, sem, psem):
    cid = lax.axis_index("c")
    sid = lax.axis_index("s")
    wid = sid * NC + cid
    b = wid // NHEAD
    h = wid % NHEAD
    qi16 = lax.broadcasted_iota(jnp.int32, (16,), 0)
    tbase = (h // 4) * (B * LIN * 4) + b * (LIN * 4) + (h % 4)

    def fire_par(ci):
        p1 = ci & 1
        q0 = pl.multiple_of(ci * CQ, CQ)
        pltpu.async_copy(par_hbm.at[b, pl.ds(q0 * 128, CQ * 128)],
                         par_v.at[p1], psem)
        pltpu.async_copy(ref_hbm.at[b, pl.ds(q0 * 2, CQ * 2)],
                         ref_v.at[p1, pl.ds(0, CQ * 2)], psem)

    def drain_par(ci):
        p1 = ci & 1
        pltpu.make_async_copy(par_hbm.at[b, pl.ds(0, CQ * 128)],
                              par_v.at[p1], psem).wait()
        pltpu.make_async_copy(ref_hbm.at[b, pl.ds(0, CQ * 2)],
                              ref_v.at[p1, pl.ds(0, CQ * 2)], psem).wait()

    def gen_and_fire(ci):
        p1 = ci & 1
        p1v = jnp.full((16,), p1, jnp.int32)
        for g in range(CQ // 16):
            qi = qi16 + (g * 16)
            base = qi * 128
            rx = plsc.load_gather(ref_v, [p1v, qi * 2])
            ry = plsc.load_gather(ref_v, [p1v, qi * 2 + 1])
            gxb = rx * float(W) - 0.5
            gyb = ry * float(H) - 0.5
            logits = [plsc.load_gather(par_v, [p1v, base + (64 + h * 4 + p)])
                      for p in range(NPOINTS)]
            m = jnp.maximum(jnp.maximum(logits[0], logits[1]),
                            jnp.maximum(logits[2], logits[3]))
            es = [jnp.exp(l - m) for l in logits]
            inv = 1.0 / (es[0] + es[1] + es[2] + es[3])
            for p in range(NPOINTS):
                ap = es[p] * inv
                ox = plsc.load_gather(par_v, [p1v, base + h * 8 + 2 * p])
                oy = plsc.load_gather(par_v,
                                      [p1v, base + h * 8 + 2 * p + 1])
                gx = gxb + ox
                gy = gyb + oy
                xt = gx.astype(jnp.int32)
                xtf = xt.astype(jnp.float32)
                xneg = xtf > gx
                x0 = jnp.where(xneg, xt - 1, xt)
                fx = gx - jnp.where(xneg, xtf - 1.0, xtf)
                yt = gy.astype(jnp.int32)
                ytf = yt.astype(jnp.float32)
                yneg = ytf > gy
                y0 = jnp.where(yneg, yt - 1, yt)
                fy = gy - jnp.where(yneg, ytf - 1.0, ytf)
                wx = [1.0 - fx, fx]
                wy = [1.0 - fy, fy]
                for dy in range(2):
                    yc = y0 + dy
                    vy = (yc >= 0) & (yc <= H - 1)
                    yci = jnp.clip(yc, 0, H - 1)
                    for dx in range(2):
                        xc = x0 + dx
                        ok = (xc >= 0) & (xc <= W - 1) & vy
                        xci = jnp.clip(xc, 0, W - 1)
                        gidx = tbase + (yci * W + xci) * 4
                        wt = ap * wx[dx] * wy[dy] * jnp.where(ok, 1.0, 0.0)
                        pos = qi * 16 + (p * 4 + dy * 2 + dx)
                        plsc.store_scatter(
                            idx_v, [p1v, lax.shift_right_logical(pos, 7),
                                    lax.bitwise_and(pos, 127)], gidx)
                        plsc.store_scatter(w_v, [p1v, pos], wt)
        for i in range(NIDX):
            pltpu.async_copy(value_hbm.at[idx_v.at[p1, i]],
                             rows_v.at[p1, pl.ds(i * 128, 128)], sem)

    fire_par(0)
    fire_par(1)

    def chunk_body(ci, carry):
        @pl.when(ci < NCHUNK)
        def _():
            drain_par(ci)
            gen_and_fire(ci)

            @pl.when(ci + 2 < NCHUNK)
            def _():
                fire_par(ci + 2)

        @pl.when(ci >= 1)
        def _():
            cj = ci - 1
            p0 = cj & 1
            q0 = pl.multiple_of(cj * CQ, CQ)
            pltpu.make_async_copy(value_hbm.at[pl.ds(0, NROW)],
                                  rows_v.at[p0], sem).wait()

            def q_body(q, carry2):
                rbase = q * 16
                wvec = w_v[p0, pl.ds(rbase, 16)]
                acc0 = jnp.zeros((16,), jnp.float32)
                acc1 = jnp.zeros((16,), jnp.float32)
                for j in range(16):
                    wj = wvec[j]
                    acc0 = acc0 + rows_v[p0, rbase + j, pl.ds(0, 16)] * wj
                    acc1 = acc1 + rows_v[p0, rbase + j, pl.ds(16, 16)] * wj
                out_v[q, pl.ds(0, 16)] = acc0
                out_v[q, pl.ds(16, 16)] = acc1
                return carry2

            lax.fori_loop(0, CQ, q_body, 0, unroll=False)
            lane0 = pl.multiple_of((h % 4) * DH, DH)
            row0 = pl.multiple_of(b * LQ + q0, CQ)

            @pl.when(h < 4)
            def _():
                pltpu.sync_copy(
                    out_v, out_lo_hbm.at[pl.ds(row0, CQ), pl.ds(lane0, DH)])

            @pl.when(h >= 4)
            def _():
                pltpu.sync_copy(
                    out_v, out_hi_hbm.at[pl.ds(row0, CQ), pl.ds(lane0, DH)])

        return carry

    lax.fori_loop(0, NCHUNK + 1, chunk_body, 0, unroll=False)


@jax.jit
def _run(tgt, uv_feature, reference_points, query_pos,
         W_off, b_off, W_attn, b_attn, W_val, b_val, W_out, b_out,
         W1, b1, W2, b2, g1, be1, g2, be2):
    f32 = jnp.float32
    uv2d = uv_feature.reshape(B * LIN, D_MODEL)
    tgt2d = tgt.reshape(B * LQ, D_MODEL)
    qp2d = query_pos.reshape(B * LQ, D_MODEL)

    W_oa = jnp.concatenate(
        [W_off, W_attn, jnp.zeros((32, D_MODEL), f32)], axis=0)
    b_oa = jnp.concatenate([b_off, b_attn, jnp.zeros((32,), f32)])
    VBLK = 4096
    QBLK = 256
    NSTEP = B * LIN // VBLK
    value2d, params = pl.pallas_call(
        _proj_body,
        grid=(NSTEP,),
        in_specs=[_rows(VBLK, D_MODEL), _full((D_MODEL, D_MODEL)),
                  _full((1, D_MODEL)),
                  _rows(QBLK, D_MODEL), _rows(QBLK, D_MODEL),
                  _full((128, D_MODEL)), _full((1, 128))],
        out_specs=(pl.BlockSpec((2, VBLK, 128), lambda i: (0, i, 0)),
                   _rows(QBLK, 128)),
        out_shape=(jax.ShapeDtypeStruct((2, B * LIN, 128), f32),
                   jax.ShapeDtypeStruct((B * LQ, 128), f32)),
    )(uv2d, W_val, b_val.reshape(1, D_MODEL),
      tgt2d, qp2d, W_oa, b_oa.reshape(1, 128))

    table = value2d.reshape(B * LIN * NHEAD, DH)
    par2d = params.reshape(B, LQ * 128)
    ref2d = reference_points.reshape(B, LQ * 2)
    mesh = plsc.VectorSubcoreMesh(core_axis_name="c", subcore_axis_name="s",
                                  num_cores=NC, num_subcores=NS)
    sampled_lo, sampled_hi = pl.kernel(
        _sc_body,
        out_type=(jax.ShapeDtypeStruct((B * LQ, 128), f32),
                  jax.ShapeDtypeStruct((B * LQ, 128), f32)),
        mesh=mesh,
        compiler_params=pltpu.CompilerParams(needs_layout_passes=False,
                                             use_tc_tiling_on_sc=False),
        scratch_types=[
            pltpu.VMEM((2, CQ * 128), f32),
            pltpu.VMEM((2, CQ * 2 + 128), f32),
            pltpu.VMEM((2, NIDX, 128), jnp.int32),
            pltpu.VMEM((2, NROW), f32),
            pltpu.VMEM((2, NROW, DH), f32),
            pltpu.VMEM((CQ, DH), f32),
            pltpu.SemaphoreType.DMA,
            pltpu.SemaphoreType.DMA,
        ],
    )(table, par2d, ref2d)

    FBLK = 512
    out = pl.pallas_call(
        _ffn_body,
        grid=(B * LQ // FBLK,),
        in_specs=[_rows(FBLK, 128), _rows(FBLK, 128),
                  _rows(FBLK, D_MODEL),
                  _full((D_MODEL, D_MODEL)), _full((1, D_MODEL)),
                  _full((DIM_FF, D_MODEL)), _full((1, DIM_FF)),
                  _full((D_MODEL, DIM_FF)), _full((1, D_MODEL)),
                  _full((1, D_MODEL)), _full((1, D_MODEL)),
                  _full((1, D_MODEL)), _full((1, D_MODEL))],
        out_specs=_rows(FBLK, D_MODEL),
        out_shape=jax.ShapeDtypeStruct((B * LQ, D_MODEL), f32),
    )(sampled_lo, sampled_hi, tgt2d,
      W_out, b_out.reshape(1, D_MODEL),
      W1, b1.reshape(1, DIM_FF),
      W2, b2.reshape(1, D_MODEL),
      g1.reshape(1, D_MODEL), be1.reshape(1, D_MODEL),
      g2.reshape(1, D_MODEL), be2.reshape(1, D_MODEL))
    return out.reshape(B, LQ, D_MODEL)


def kernel(tgt, uv_feature, reference_points, query_pos,
           input_spatial_shapes, input_level_start_index,
           W_off, b_off, W_attn, b_attn, W_val, b_val, W_out, b_out,
           W1, b1, W2, b2, g1, be1, g2, be2):
    return _run(tgt, uv_feature, reference_points, query_pos,
                W_off, b_off, W_attn, b_attn, W_val, b_val, W_out, b_out,
                W1, b1, W2, b2, g1, be1, g2, be2)


# speedup vs baseline: 1.0749x; 1.0080x over previous
import functools

import jax
import jax.numpy as jnp
from jax import lax
from jax.experimental import pallas as pl
from jax.experimental.pallas import tpu as pltpu
from jax.experimental.pallas import tpu_sc as plsc

D_MODEL = 256
DIM_FF = 2048
NHEAD = 8
DH = D_MODEL // NHEAD
NPOINTS = 4
B = 4
LQ = 1024
H = 128
W = 128
LIN = H * W

NC = 2
NS = 16
NW = NC * NS

CQ = 64
NCHUNK = LQ // CQ
NROW = CQ * NPOINTS * 4
NIDX = NROW // 128


def _nt_dot(x, w):
    return lax.dot_general(x, w, (((1,), (1,)), ((), ())),
                           preferred_element_type=jnp.float32)


def _proj_body(x_ref, w_ref, b_ref, t_ref, p_ref, wq_ref, bq_ref,
               o_ref, oq_ref):
    r = _nt_dot(x_ref[...], w_ref[...]) + b_ref[...]
    o_ref[0] = r[:, :128]
    o_ref[1] = r[:, 128:]
    q = t_ref[...] + p_ref[...]
    oq_ref[...] = _nt_dot(q, wq_ref[...]) + bq_ref[...]


def _layernorm(x, g, b):
    mu = jnp.mean(x, axis=-1, keepdims=True)
    xc = x - mu
    var = jnp.mean(xc * xc, axis=-1, keepdims=True)
    return xc * lax.rsqrt(var + 1e-5) * g + b


def _ffn_body(s_ref, s2_ref, t_ref, wo_ref, bo_ref, w1_ref, b1_ref, w2_ref,
              b2_ref, g1_ref, be1_ref, g2_ref, be2_ref, o_ref):
    s = jnp.concatenate([s_ref[...], s2_ref[...]], axis=-1)
    tgt2 = _nt_dot(s, wo_ref[...]) + bo_ref[...]
    x = _layernorm(t_ref[...] + tgt2, g1_ref[...], be1_ref[...])
    ff = jnp.maximum(_nt_dot(x, w1_ref[...]) + b1_ref[...], 0.0)
    ff2 = _nt_dot(ff, w2_ref[...]) + b2_ref[...]
    o_ref[...] = _layernorm(x + ff2, g2_ref[...], be2_ref[...])


def _full(shape):
    return pl.BlockSpec(shape, lambda i: (0,) * len(shape))


def _rows(blk, cols):
    return pl.BlockSpec((blk, cols), lambda i: (i, 0))


def _sc_body(value_hbm, par_hbm, ref_hbm, out_lo_hbm, out_hi_hbm,
             par_v, ref_v, idx_v, w_v, rows_v, out_v, sem, psem):
    cid = lax.axis_index("c")
    sid = lax.axis_index("s")
    wid = sid * NC + cid
    b = wid // NHEAD
    h = wid % NHEAD
    qi16 = lax.broadcasted_iota(jnp.int32, (16,), 0)
    tbase = (h // 4) * (B * LIN * 4) + b * (LIN * 4) + (h % 4)

    def fire_par(ci):
        p1 = ci & 1
        q0 = pl.multiple_of(ci * CQ, CQ)
        pltpu.async_copy(par_hbm.at[b, pl.ds(q0 * 128, CQ * 128)],
                         par_v.at[p1], psem)
        pltpu.async_copy(ref_hbm.at[b, pl.ds(q0 * 2, CQ * 2)],
                         ref_v.at[p1, pl.ds(0, CQ * 2)], psem)

    def drain_par(ci):
        p1 = ci & 1
        pltpu.make_async_copy(par_hbm.at[b, pl.ds(0, CQ * 128)],
                              par_v.at[p1], psem).wait()
        pltpu.make_async_copy(ref_hbm.at[b, pl.ds(0, CQ * 2)],
                              ref_v.at[p1, pl.ds(0, CQ * 2)], psem).wait()

    def gen_and_fire(ci):
        p1 = ci & 1
        p1v = jnp.full((16,), p1, jnp.int32)
        for g in range(CQ // 16):
            qi = qi16 + (g * 16)
            base = qi * 128
            rx = plsc.load_gather(ref_v, [p1v, qi * 2])
            ry = plsc.load_gather(ref_v, [p1v, qi * 2 + 1])
            gxb = rx * float(W) - 0.5
            gyb = ry * float(H) - 0.5
            logits = [plsc.load_gather(par_v, [p1v, base + (64 + h * 4 + p)])
                      for p in range(NPOINTS)]
            m = jnp.maximum(jnp.maximum(logits[0], logits[1]),
                            jnp.maximum(logits[2], logits[3]))
            es = [jnp.exp(l - m) for l in logits]
            inv = 1.0 / (es[0] + es[1] + es[2] + es[3])
            for p in range(NPOINTS):
                ap = es[p] * inv
                ox = plsc.load_gather(par_v, [p1v, base + h * 8 + 2 * p])
                oy = plsc.load_gather(par_v,
                                      [p1v, base + h * 8 + 2 * p + 1])
                gx = gxb + ox
                gy = gyb + oy
                xt = gx.astype(jnp.int32)
                xtf = xt.astype(jnp.float32)
                xneg = xtf > gx
                x0 = jnp.where(xneg, xt - 1, xt)
                fx = gx - jnp.where(xneg, xtf - 1.0, xtf)
                yt = gy.astype(jnp.int32)
                ytf = yt.astype(jnp.float32)
                yneg = ytf > gy
                y0 = jnp.where(yneg, yt - 1, yt)
                fy = gy - jnp.where(yneg, ytf - 1.0, ytf)
                wx = [1.0 - fx, fx]
                wy = [1.0 - fy, fy]
                for dy in range(2):
                    yc = y0 + dy
                    vy = (yc >= 0) & (yc <= H - 1)
                    yci = jnp.clip(yc, 0, H - 1)
                    for dx in range(2):
                        xc = x0 + dx
                        ok = (xc >= 0) & (xc <= W - 1) & vy
                        xci = jnp.clip(xc, 0, W - 1)
                        gidx = tbase + (yci * W + xci) * 4
                        wt = ap * wx[dx] * wy[dy] * jnp.where(ok, 1.0, 0.0)
                        pos = qi * 16 + (p * 4 + dy * 2 + dx)
                        plsc.store_scatter(
                            idx_v, [p1v, lax.shift_right_logical(pos, 7),
                                    lax.bitwise_and(pos, 127)], gidx)
                        plsc.store_scatter(w_v, [p1v, pos], wt)
        for i in range(NIDX):
            pltpu.async_copy(value_hbm.at[idx_v.at[p1, i]],
                             rows_v.at[p1, pl.ds(i * 128, 128)], sem)

    fire_par(0)
    fire_par(1)

    def chunk_body(ci, carry):
        @pl.when(ci < NCHUNK)
        def _():
            drain_par(ci)
            gen_and_fire(ci)

            @pl.when(ci + 2 < NCHUNK)
            def _():
                fire_par(ci + 2)

        @pl.when(ci >= 1)
        def _():
            cj = ci - 1
            p0 = cj & 1
            q0 = pl.multiple_of(cj * CQ, CQ)
            pltpu.make_async_copy(value_hbm.at[pl.ds(0, NROW)],
                                  rows_v.at[p0], sem).wait()

            def q_body(q, carry2):
                rbase = q * 16
                wvec = w_v[p0, pl.ds(rbase, 16)]
                acc0 = jnp.zeros((16,), jnp.float32)
                acc1 = jnp.zeros((16,), jnp.float32)
                for j in range(16):
                    wj = wvec[j]
                    acc0 = acc0 + rows_v[p0, rbase + j, pl.ds(0, 16)] * wj
                    acc1 = acc1 + rows_v[p0, rbase + j, pl.ds(16, 16)] * wj
                out_v[q, pl.ds(0, 16)] = acc0
                out_v[q, pl.ds(16, 16)] = acc1
                return carry2

            lax.fori_loop(0, CQ, q_body, 0, unroll=False)
            lane0 = pl.multiple_of((h % 4) * DH, DH)
            row0 = pl.multiple_of(b * LQ + q0, CQ)

            @pl.when(h < 4)
            def _():
                pltpu.sync_copy(
                    out_v, out_lo_hbm.at[pl.ds(row0, CQ), pl.ds(lane0, DH)])

            @pl.when(h >= 4)
            def _():
                pltpu.sync_copy(
                    out_v, out_hi_hbm.at[pl.ds(row0, CQ), pl.ds(lane0, DH)])

        return carry

    lax.fori_loop(0, NCHUNK + 1, chunk_body, 0, unroll=False)


@jax.jit
def _run(tgt, uv_feature, reference_points, query_pos,
         W_off, b_off, W_attn, b_attn, W_val, b_val, W_out, b_out,
         W1, b1, W2, b2, g1, be1, g2, be2):
    f32 = jnp.float32
    uv2d = uv_feature.reshape(B * LIN, D_MODEL)
    tgt2d = tgt.reshape(B * LQ, D_MODEL)
    qp2d = query_pos.reshape(B * LQ, D_MODEL)

    W_oa = jnp.concatenate(
        [W_off, W_attn, jnp.zeros((32, D_MODEL), f32)], axis=0)
    b_oa = jnp.concatenate([b_off, b_attn, jnp.zeros((32,), f32)])
    VBLK = 8192
    QBLK = 512
    NSTEP = B * LIN // VBLK
    value2d, params = pl.pallas_call(
        _proj_body,
        grid=(NSTEP,),
        in_specs=[_rows(VBLK, D_MODEL), _full((D_MODEL, D_MODEL)),
                  _full((1, D_MODEL)),
                  _rows(QBLK, D_MODEL), _rows(QBLK, D_MODEL),
                  _full((128, D_MODEL)), _full((1, 128))],
        out_specs=(pl.BlockSpec((2, VBLK, 128), lambda i: (0, i, 0)),
                   _rows(QBLK, 128)),
        out_shape=(jax.ShapeDtypeStruct((2, B * LIN, 128), f32),
                   jax.ShapeDtypeStruct((B * LQ, 128), f32)),
    )(uv2d, W_val, b_val.reshape(1, D_MODEL),
      tgt2d, qp2d, W_oa, b_oa.reshape(1, 128))

    table = value2d.reshape(B * LIN * NHEAD, DH)
    par2d = params.reshape(B, LQ * 128)
    ref2d = reference_points.reshape(B, LQ * 2)
    mesh = plsc.VectorSubcoreMesh(core_axis_name="c", subcore_axis_name="s",
                                  num_cores=NC, num_subcores=NS)
    sampled_lo, sampled_hi = pl.kernel(
        _sc_body,
        out_type=(jax.ShapeDtypeStruct((B * LQ, 128), f32),
                  jax.ShapeDtypeStruct((B * LQ, 128), f32)),
        mesh=mesh,
        compiler_params=pltpu.CompilerParams(needs_layout_passes=False,
                                             use_tc_tiling_on_sc=False),
        scratch_types=[
            pltpu.VMEM((2, CQ * 128), f32),
            pltpu.VMEM((2, CQ * 2 + 128), f32),
            pltpu.VMEM((2, NIDX, 128), jnp.int32),
            pltpu.VMEM((2, NROW), f32),
            pltpu.VMEM((2, NROW, DH), f32),
            pltpu.VMEM((CQ, DH), f32),
            pltpu.SemaphoreType.DMA,
            pltpu.SemaphoreType.DMA,
        ],
    )(table, par2d, ref2d)

    FBLK = 512
    out = pl.pallas_call(
        _ffn_body,
        grid=(B * LQ // FBLK,),
        in_specs=[_rows(FBLK, 128), _rows(FBLK, 128),
                  _rows(FBLK, D_MODEL),
                  _full((D_MODEL, D_MODEL)), _full((1, D_MODEL)),
                  _full((DIM_FF, D_MODEL)), _full((1, DIM_FF)),
                  _full((D_MODEL, DIM_FF)), _full((1, D_MODEL)),
                  _full((1, D_MODEL)), _full((1, D_MODEL)),
                  _full((1, D_MODEL)), _full((1, D_MODEL))],
        out_specs=_rows(FBLK, D_MODEL),
        out_shape=jax.ShapeDtypeStruct((B * LQ, D_MODEL), f32),
    )(sampled_lo, sampled_hi, tgt2d,
      W_out, b_out.reshape(1, D_MODEL),
      W1, b1.reshape(1, DIM_FF),
      W2, b2.reshape(1, D_MODEL),
      g1.reshape(1, D_MODEL), be1.reshape(1, D_MODEL),
      g2.reshape(1, D_MODEL), be2.reshape(1, D_MODEL))
    return out.reshape(B, LQ, D_MODEL)


def kernel(tgt, uv_feature, reference_points, query_pos,
           input_spatial_shapes, input_level_start_index,
           W_off, b_off, W_attn, b_attn, W_val, b_val, W_out, b_out,
           W1, b1, W2, b2, g1, be1, g2, be2):
    return _run(tgt, uv_feature, reference_points, query_pos,
                W_off, b_off, W_attn, b_attn, W_val, b_val, W_out, b_out,
                W1, b1, W2, b2, g1, be1, g2, be2)


# speedup vs baseline: 1.0924x; 1.0163x over previous
import functools

import jax
import jax.numpy as jnp
from jax import lax
from jax.experimental import pallas as pl
from jax.experimental.pallas import tpu as pltpu
from jax.experimental.pallas import tpu_sc as plsc

D_MODEL = 256
DIM_FF = 2048
NHEAD = 8
DH = D_MODEL // NHEAD
NPOINTS = 4
B = 4
LQ = 1024
H = 128
W = 128
LIN = H * W

NC = 2
NS = 16
NW = NC * NS

CQ = 64
NCHUNK = LQ // CQ
NROW = CQ * NPOINTS * 4
NIDX = NROW // 128


def _nt_dot(x, w):
    return lax.dot_general(x, w, (((1,), (1,)), ((), ())),
                           preferred_element_type=jnp.float32)


def _proj_body(x_ref, w_ref, b_ref, t_ref, p_ref, wq_ref, bq_ref,
               o_ref, oq_ref):
    r = _nt_dot(x_ref[...], w_ref[...]) + b_ref[...]
    o_ref[0] = r[:, :128]
    o_ref[1] = r[:, 128:]
    q = t_ref[...] + p_ref[...]
    oq_ref[...] = _nt_dot(q, wq_ref[...]) + bq_ref[...]


def _layernorm(x, g, b):
    mu = jnp.mean(x, axis=-1, keepdims=True)
    xc = x - mu
    var = jnp.mean(xc * xc, axis=-1, keepdims=True)
    return xc * lax.rsqrt(var + 1e-5) * g + b


def _ffn_body(s_ref, s2_ref, t_ref, wo_ref, bo_ref, w1_ref, b1_ref, w2_ref,
              b2_ref, g1_ref, be1_ref, g2_ref, be2_ref, o_ref):
    s = jnp.concatenate([s_ref[...], s2_ref[...]], axis=-1)
    tgt2 = _nt_dot(s, wo_ref[...]) + bo_ref[...]
    x = _layernorm(t_ref[...] + tgt2, g1_ref[...], be1_ref[...])
    ff = jnp.maximum(_nt_dot(x, w1_ref[...]) + b1_ref[...], 0.0)
    ff2 = _nt_dot(ff, w2_ref[...]) + b2_ref[...]
    o_ref[...] = _layernorm(x + ff2, g2_ref[...], be2_ref[...])


def _full(shape):
    return pl.BlockSpec(shape, lambda i: (0,) * len(shape))


def _rows(blk, cols):
    return pl.BlockSpec((blk, cols), lambda i: (i, 0))


def _sc_body(value_hbm, par_hbm, ref_hbm, out_lo_hbm, out_hi_hbm,
             par_v, ref_v, idx_v, w_v, rows_v, out_v, sem, psem):
    cid = lax.axis_index("c")
    sid = lax.axis_index("s")
    wid = sid * NC + cid
    b = wid // NHEAD
    h = wid % NHEAD
    qi16 = lax.broadcasted_iota(jnp.int32, (16,), 0)
    tbase = (h // 4) * (B * LIN * 4) + b * (LIN * 4) + (h % 4)

    def fire_par(ci):
        p1 = ci & 1
        q0 = pl.multiple_of(ci * CQ, CQ)
        pltpu.async_copy(par_hbm.at[b, pl.ds(q0 * 128, CQ * 128)],
                         par_v.at[p1], psem)
        pltpu.async_copy(ref_hbm.at[b, pl.ds(q0 * 2, CQ * 2)],
                         ref_v.at[p1, pl.ds(0, CQ * 2)], psem)

    def drain_par(ci):
        p1 = ci & 1
        pltpu.make_async_copy(par_hbm.at[b, pl.ds(0, CQ * 128)],
                              par_v.at[p1], psem).wait()
        pltpu.make_async_copy(ref_hbm.at[b, pl.ds(0, CQ * 2)],
                              ref_v.at[p1, pl.ds(0, CQ * 2)], psem).wait()

    def gen_and_fire(ci):
        p1 = ci & 1
        p1v = jnp.full((16,), p1, jnp.int32)
        for g in range(CQ // 16):
            qi = qi16 + (g * 16)
            base = qi * 128
            rx = plsc.load_gather(ref_v, [p1v, qi * 2])
            ry = plsc.load_gather(ref_v, [p1v, qi * 2 + 1])
            gxb = rx * float(W) - 0.5
            gyb = ry * float(H) - 0.5
            logits = [plsc.load_gather(par_v, [p1v, base + (64 + h * 4 + p)])
                      for p in range(NPOINTS)]
            m = jnp.maximum(jnp.maximum(logits[0], logits[1]),
                            jnp.maximum(logits[2], logits[3]))
            es = [jnp.exp(l - m) for l in logits]
            inv = 1.0 / (es[0] + es[1] + es[2] + es[3])
            for p in range(NPOINTS):
                ap = es[p] * inv
                ox = plsc.load_gather(par_v, [p1v, base + h * 8 + 2 * p])
                oy = plsc.load_gather(par_v,
                                      [p1v, base + h * 8 + 2 * p + 1])
                gx = gxb + ox
                gy = gyb + oy
                xt = gx.astype(jnp.int32)
                xtf = xt.astype(jnp.float32)
                xneg = xtf > gx
                x0 = jnp.where(xneg, xt - 1, xt)
                fx = gx - jnp.where(xneg, xtf - 1.0, xtf)
                yt = gy.astype(jnp.int32)
                ytf = yt.astype(jnp.float32)
                yneg = ytf > gy
                y0 = jnp.where(yneg, yt - 1, yt)
                fy = gy - jnp.where(yneg, ytf - 1.0, ytf)
                wx = [1.0 - fx, fx]
                wy = [1.0 - fy, fy]
                for dy in range(2):
                    yc = y0 + dy
                    vy = (yc >= 0) & (yc <= H - 1)
                    yci = jnp.clip(yc, 0, H - 1)
                    for dx in range(2):
                        xc = x0 + dx
                        ok = (xc >= 0) & (xc <= W - 1) & vy
                        xci = jnp.clip(xc, 0, W - 1)
                        gidx = tbase + (yci * W + xci) * 4
                        wt = ap * wx[dx] * wy[dy] * jnp.where(ok, 1.0, 0.0)
                        pos = qi * 16 + (p * 4 + dy * 2 + dx)
                        plsc.store_scatter(
                            idx_v, [p1v, lax.shift_right_logical(pos, 7),
                                    lax.bitwise_and(pos, 127)], gidx)
                        plsc.store_scatter(w_v, [p1v, pos], wt)
        for i in range(NIDX):
            pltpu.async_copy(value_hbm.at[idx_v.at[p1, i]],
                             rows_v.at[p1, pl.ds(i * 128, 128)], sem)

    fire_par(0)
    fire_par(1)

    def chunk_body(ci, carry):
        @pl.when(ci < NCHUNK)
        def _():
            drain_par(ci)
            gen_and_fire(ci)

            @pl.when(ci + 2 < NCHUNK)
            def _():
                fire_par(ci + 2)

        @pl.when(ci >= 1)
        def _():
            cj = ci - 1
            p0 = cj & 1
            q0 = pl.multiple_of(cj * CQ, CQ)
            pltpu.make_async_copy(value_hbm.at[pl.ds(0, NROW)],
                                  rows_v.at[p0], sem).wait()

            def q_body(q, carry2):
                rbase = q * 16
                wvec = w_v[p0, pl.ds(rbase, 16)]
                acc0 = jnp.zeros((16,), jnp.float32)
                acc1 = jnp.zeros((16,), jnp.float32)
                for j in range(16):
                    wj = wvec[j]
                    acc0 = acc0 + rows_v[p0, rbase + j, pl.ds(0, 16)] * wj
                    acc1 = acc1 + rows_v[p0, rbase + j, pl.ds(16, 16)] * wj
                out_v[q, pl.ds(0, 16)] = acc0
                out_v[q, pl.ds(16, 16)] = acc1
                return carry2

            lax.fori_loop(0, CQ, q_body, 0, unroll=False)
            lane0 = pl.multiple_of((h % 4) * DH, DH)
            row0 = pl.multiple_of(b * LQ + q0, CQ)

            @pl.when(h < 4)
            def _():
                pltpu.sync_copy(
                    out_v, out_lo_hbm.at[pl.ds(row0, CQ), pl.ds(lane0, DH)])

            @pl.when(h >= 4)
            def _():
                pltpu.sync_copy(
                    out_v, out_hi_hbm.at[pl.ds(row0, CQ), pl.ds(lane0, DH)])

        return carry

    lax.fori_loop(0, NCHUNK + 1, chunk_body, 0, unroll=False)


@jax.jit
def _run(tgt, uv_feature, reference_points, query_pos,
         W_off, b_off, W_attn, b_attn, W_val, b_val, W_out, b_out,
         W1, b1, W2, b2, g1, be1, g2, be2):
    f32 = jnp.float32
    uv2d = uv_feature.reshape(B * LIN, D_MODEL)
    tgt2d = tgt.reshape(B * LQ, D_MODEL)
    qp2d = query_pos.reshape(B * LQ, D_MODEL)

    W_oa = jnp.concatenate(
        [W_off, W_attn, jnp.zeros((32, D_MODEL), f32)], axis=0)
    b_oa = jnp.concatenate([b_off, b_attn, jnp.zeros((32,), f32)])
    VBLK = 8192
    QBLK = 512
    NSTEP = B * LIN // VBLK
    value2d, params = pl.pallas_call(
        _proj_body,
        grid=(NSTEP,),
        in_specs=[_rows(VBLK, D_MODEL), _full((D_MODEL, D_MODEL)),
                  _full((1, D_MODEL)),
                  _rows(QBLK, D_MODEL), _rows(QBLK, D_MODEL),
                  _full((128, D_MODEL)), _full((1, 128))],
        out_specs=(pl.BlockSpec((2, VBLK, 128), lambda i: (0, i, 0)),
                   _rows(QBLK, 128)),
        out_shape=(jax.ShapeDtypeStruct((2, B * LIN, 128), f32),
                   jax.ShapeDtypeStruct((B * LQ, 128), f32)),
    )(uv2d, W_val, b_val.reshape(1, D_MODEL),
      tgt2d, qp2d, W_oa, b_oa.reshape(1, 128))

    table = value2d.reshape(B * LIN * NHEAD, DH)
    par2d = params.reshape(B, LQ * 128)
    ref2d = reference_points.reshape(B, LQ * 2)
    mesh = plsc.VectorSubcoreMesh(core_axis_name="c", subcore_axis_name="s",
                                  num_cores=NC, num_subcores=NS)
    sampled_lo, sampled_hi = pl.kernel(
        _sc_body,
        out_type=(jax.ShapeDtypeStruct((B * LQ, 128), f32),
                  jax.ShapeDtypeStruct((B * LQ, 128), f32)),
        mesh=mesh,
        compiler_params=pltpu.CompilerParams(needs_layout_passes=False,
                                             use_tc_tiling_on_sc=False),
        scratch_types=[
            pltpu.VMEM((2, CQ * 128), f32),
            pltpu.VMEM((2, CQ * 2 + 128), f32),
            pltpu.VMEM((2, NIDX, 128), jnp.int32),
            pltpu.VMEM((2, NROW), f32),
            pltpu.VMEM((2, NROW, DH), f32),
            pltpu.VMEM((CQ, DH), f32),
            pltpu.SemaphoreType.DMA,
            pltpu.SemaphoreType.DMA,
        ],
    )(table, par2d, ref2d)

    FBLK = 1024
    out = pl.pallas_call(
        _ffn_body,
        grid=(B * LQ // FBLK,),
        in_specs=[_rows(FBLK, 128), _rows(FBLK, 128),
                  _rows(FBLK, D_MODEL),
                  _full((D_MODEL, D_MODEL)), _full((1, D_MODEL)),
                  _full((DIM_FF, D_MODEL)), _full((1, DIM_FF)),
                  _full((D_MODEL, DIM_FF)), _full((1, D_MODEL)),
                  _full((1, D_MODEL)), _full((1, D_MODEL)),
                  _full((1, D_MODEL)), _full((1, D_MODEL))],
        out_specs=_rows(FBLK, D_MODEL),
        out_shape=jax.ShapeDtypeStruct((B * LQ, D_MODEL), f32),
    )(sampled_lo, sampled_hi, tgt2d,
      W_out, b_out.reshape(1, D_MODEL),
      W1, b1.reshape(1, DIM_FF),
      W2, b2.reshape(1, D_MODEL),
      g1.reshape(1, D_MODEL), be1.reshape(1, D_MODEL),
      g2.reshape(1, D_MODEL), be2.reshape(1, D_MODEL))
    return out.reshape(B, LQ, D_MODEL)


def kernel(tgt, uv_feature, reference_points, query_pos,
           input_spatial_shapes, input_level_start_index,
           W_off, b_off, W_attn, b_attn, W_val, b_val, W_out, b_out,
           W1, b1, W2, b2, g1, be1, g2, be2):
    return _run(tgt, uv_feature, reference_points, query_pos,
                W_off, b_off, W_attn, b_attn, W_val, b_val, W_out, b_out,
                W1, b1, W2, b2, g1, be1, g2, be2)
